# Initial kernel scaffold; baseline (speedup 1.0000x reference)
import jax
import jax.numpy as jnp
from jax.experimental import pallas as pl


def kernel(x, table):
    raise NotImplementedError("write your pallas kernel here")


# speedup vs baseline: 5.4003x; 5.4003x over previous
import functools

import jax
import jax.numpy as jnp
from jax import lax
from jax.experimental import pallas as pl
from jax.experimental.pallas import tpu as pltpu
from jax.experimental.pallas import tpu_sc as plsc

NC = 2
NS = 16
NW = NC * NS
L = 16

ROWS, COLS = 16384, 200
D = 4
N_TOTAL = ROWS * COLS
PER_W = N_TOTAL // NW
CHUNK = 6400
NCHUNK = PER_W // CHUNK
BLOCKS = CHUNK // L


def _take16(src, idx):
  return jnp.take_along_axis(src, idx, axis=0, mode="promise_in_bounds")


def _emb_body(x_hbm, tab_hbm, out_hbm, idx_v, out_v, tab_v):
  wid = lax.axis_index("s") * NC + lax.axis_index("c")
  base = wid * PER_W

  pltpu.sync_copy(tab_hbm, tab_v)
  tbl = [tab_v[pl.ds(L * k, L)] for k in range(D)]
  lane = lax.iota(jnp.int32, L)
  inv = [lane * D + k for k in range(D)]

  def chunk_body(c, carry):
    cbase = base + c * CHUNK
    pltpu.sync_copy(x_hbm.at[pl.ds(cbase, CHUNK)], idx_v)

    def blk(m, carry2):
      vidx = idx_v[pl.ds(m * L, L)]
      off = m * (L * D)
      for k in range(D):
        col = _take16(tbl[k], vidx)
        plsc.store_scatter(out_v, [inv[k] + off], col)
      return carry2

    lax.fori_loop(0, BLOCKS, blk, 0)
    pltpu.sync_copy(out_v, out_hbm.at[pl.ds(cbase * D, CHUNK * D)])
    return carry

  lax.fori_loop(0, NCHUNK, chunk_body, 0)


@functools.partial(jax.jit, donate_argnums=())
def _emb_call(x_flat, tab_t):
  mesh = plsc.VectorSubcoreMesh(core_axis_name="c", subcore_axis_name="s")
  f = pl.kernel(
      _emb_body,
      out_type=jax.ShapeDtypeStruct((N_TOTAL * D,), jnp.float32),
      mesh=mesh,
      scratch_types=[
          pltpu.VMEM((CHUNK,), jnp.int32),
          pltpu.VMEM((CHUNK * D,), jnp.float32),
          pltpu.VMEM((D * L,), jnp.float32),
      ],
      compiler_params=pltpu.CompilerParams(needs_layout_passes=False),
  )
  return f(x_flat, tab_t)


def kernel(x, table):
  x_flat = x.reshape(-1)
  tab_t = jnp.zeros((D, L), jnp.float32).at[:, :table.shape[0]].set(table.T)
  out = _emb_call(x_flat, tab_t.reshape(-1))
  return out.reshape(ROWS, COLS, D)


# speedup vs baseline: 5.4563x; 1.0104x over previous
import functools

import jax
import jax.numpy as jnp
from jax import lax
from jax.experimental import pallas as pl
from jax.experimental.pallas import tpu as pltpu
from jax.experimental.pallas import tpu_sc as plsc

NC = 2
NS = 16
NW = NC * NS
L = 16

ROWS, COLS = 16384, 200
D = 4
N_TOTAL = ROWS * COLS
PER_W = N_TOTAL // NW
CHUNK = 6400
NCHUNK = PER_W // CHUNK
BLOCKS = CHUNK // L


def _take16(src, idx):
  return jnp.take_along_axis(src, idx, axis=0, mode="promise_in_bounds")


def _emb_body(x_hbm, tab_hbm, out_hbm, idx_v, out_v, tab_v):
  wid = lax.axis_index("s") * NC + lax.axis_index("c")
  base = wid * PER_W

  pltpu.sync_copy(tab_hbm, tab_v)
  tbl = [tab_v[pl.ds(L * k, L)] for k in range(D)]
  lane = lax.iota(jnp.int32, L)
  inv = [lane * D + k for k in range(D)]

  def chunk_body(c, carry):
    cbase = base + c * CHUNK
    pltpu.sync_copy(x_hbm.at[pl.ds(cbase, CHUNK)], idx_v)

    @plsc.parallel_loop(0, BLOCKS, 1, unroll=8)
    def _blk(m):
      vidx = idx_v[pl.ds(m * L, L)]
      off = m * (L * D)
      for k in range(D):
        col = _take16(tbl[k], vidx)
        plsc.store_scatter(out_v, [inv[k] + off], col)
    pltpu.sync_copy(out_v, out_hbm.at[pl.ds(cbase * D, CHUNK * D)])
    return carry

  lax.fori_loop(0, NCHUNK, chunk_body, 0)


@functools.partial(jax.jit, donate_argnums=())
def _emb_call(x_flat, tab_t):
  mesh = plsc.VectorSubcoreMesh(core_axis_name="c", subcore_axis_name="s")
  f = pl.kernel(
      _emb_body,
      out_type=jax.ShapeDtypeStruct((N_TOTAL * D,), jnp.float32),
      mesh=mesh,
      scratch_types=[
          pltpu.VMEM((CHUNK,), jnp.int32),
          pltpu.VMEM((CHUNK * D,), jnp.float32),
          pltpu.VMEM((D * L,), jnp.float32),
      ],
      compiler_params=pltpu.CompilerParams(needs_layout_passes=False),
  )
  return f(x_flat, tab_t)


def kernel(x, table):
  x_flat = x.reshape(-1)
  tab_t = jnp.zeros((D, L), jnp.float32).at[:, :table.shape[0]].set(table.T)
  out = _emb_call(x_flat, tab_t.reshape(-1))
  return out.reshape(ROWS, COLS, D)


# speedup vs baseline: 170.0106x; 31.1587x over previous
import functools

import jax
import jax.numpy as jnp
from jax import lax
from jax.experimental import pallas as pl
from jax.experimental.pallas import tpu as pltpu
from jax.experimental.pallas import tpu_sc as plsc

NC = 2
NS = 16
NW = NC * NS
L = 16

ROWS, COLS = 16384, 200
D = 4
N_TOTAL = ROWS * COLS
NJT = COLS // 8
NIT = ROWS // 128
IT_PER_W = NIT // NW
XBLK = IT_PER_W * 8 * 128
OVBLK = 8 * IT_PER_W * D * 128


def _take16(src, idx):
  return jnp.take_along_axis(src, idx, axis=0, mode="promise_in_bounds")


def _emb_body(x_hbm, tab_hbm, out_hbm, xv, ov, tab_v, sem):
  wid = lax.axis_index("s") * NC + lax.axis_index("c")
  it0 = wid * IT_PER_W

  pltpu.sync_copy(tab_hbm, tab_v)
  tbl = [tab_v[pl.ds(L * k, L)] for k in range(D)]

  def jt_body(jt, carry):
    pltpu.sync_copy(x_hbm.at[pl.ds((jt * NIT + it0) * 1024, XBLK)], xv)

    @plsc.parallel_loop(0, XBLK // L, 1, unroll=4)
    def _blk(m):
      vidx = xv[pl.ds(m * L, L)]
      dit = m >> 6
      s = (m >> 3) & 7
      l0 = m & 7
      base = (s * (IT_PER_W * D) + dit * D) * 128 + l0 * L
      for k in range(D):
        ov[pl.ds(base + k * 128, L)] = _take16(tbl[k], vidx)

    copies = []
    for s in range(8):
      dst0 = ((jt * 8 + s) * 128 + it0) * 512
      copies.append(
          pltpu.async_copy(
              ov.at[pl.ds(s * 2048, 2048)],
              out_hbm.at[pl.ds(dst0, 2048)],
              sem,
          )
      )
    for c in copies:
      c.wait()
    return carry

  lax.fori_loop(0, NJT, jt_body, 0)


@jax.jit
def _emb_call(x_flat, tab_t):
  mesh = plsc.VectorSubcoreMesh(core_axis_name="c", subcore_axis_name="s")
  f = pl.kernel(
      _emb_body,
      out_type=jax.ShapeDtypeStruct((N_TOTAL * D,), jnp.float32),
      mesh=mesh,
      scratch_types=[
          pltpu.VMEM((XBLK,), jnp.int32),
          pltpu.VMEM((OVBLK,), jnp.float32),
          pltpu.VMEM((D * L,), jnp.float32),
          pltpu.SemaphoreType.DMA,
      ],
      compiler_params=pltpu.CompilerParams(needs_layout_passes=False),
  )
  return f(x_flat, tab_t)


def kernel(x, table):
  x_q = (
      x.T.reshape(NJT, 8, NIT, 128)
      .transpose(0, 2, 1, 3)
      .reshape(-1)
  )
  tab_t = jnp.zeros((D, L), jnp.float32).at[:, : table.shape[0]].set(table.T)
  out_q = _emb_call(x_q, tab_t.reshape(-1))
  return (
      out_q.reshape(COLS, NIT, D, 128)
      .transpose(1, 3, 0, 2)
      .reshape(ROWS, COLS, D)
  )


# speedup vs baseline: 238.0631x; 1.4003x over previous
import functools

import jax
import jax.numpy as jnp
from jax import lax
from jax.experimental import pallas as pl
from jax.experimental.pallas import tpu as pltpu
from jax.experimental.pallas import tpu_sc as plsc

NC = 2
NS = 16
NW = NC * NS
L = 16

ROWS, COLS = 16384, 200
D = 4
N_TOTAL = ROWS * COLS
NJT = COLS // 8
NIT = ROWS // 128
IT_PER_W = NIT // NW
XBLK = IT_PER_W * 8 * 128
OVBLK = 8 * IT_PER_W * D * 128


def _take16(src, idx):
  return jnp.take_along_axis(src, idx, axis=0, mode="promise_in_bounds")


def _emb_body(x_hbm, tab_hbm, out_hbm, xv, ov, tab_v, sem_in, sem_out):
  wid = lax.axis_index("s") * NC + lax.axis_index("c")
  it0 = wid * IT_PER_W

  pltpu.sync_copy(tab_hbm, tab_v)
  tbl = [tab_v[pl.ds(L * k, L)] for k in range(D)]

  def issue_in(jt, b):
    pltpu.async_copy(
        x_hbm.at[pl.ds((jt * NIT + it0) * 1024, XBLK)], xv.at[b], sem_in
    )

  def wait_in(b):
    pltpu.make_async_copy(
        x_hbm.at[pl.ds(0, XBLK)], xv.at[b], sem_in
    ).wait()

  def drain_out(b):
    pltpu.make_async_copy(
        ov.at[b], out_hbm.at[pl.ds(0, OVBLK)], sem_out.at[b]
    ).wait()

  issue_in(0, 0)

  def jt_body(jt, carry):
    b = jt & 1
    wait_in(b)

    @pl.when(jt < NJT - 1)
    def _():
      issue_in(jt + 1, 1 - b)

    @pl.when(jt >= 2)
    def _():
      drain_out(b)

    @plsc.parallel_loop(0, XBLK // L, 1, unroll=4)
    def _blk(m):
      vidx = xv[b, pl.ds(m * L, L)]
      dit = m >> 6
      s = (m >> 3) & 7
      l0 = m & 7
      base = (s * (IT_PER_W * D) + dit * D) * 128 + l0 * L
      for k in range(D):
        ov[b, pl.ds(base + k * 128, L)] = _take16(tbl[k], vidx)

    for s in range(8):
      dst0 = ((jt * 8 + s) * 128 + it0) * 512
      pltpu.async_copy(
          ov.at[b, pl.ds(s * 2048, 2048)],
          out_hbm.at[pl.ds(dst0, 2048)],
          sem_out.at[b],
      )
    return carry

  lax.fori_loop(0, NJT, jt_body, 0)
  drain_out(1)
  drain_out(0)


@jax.jit
def _emb_call(x_flat, tab_t):
  mesh = plsc.VectorSubcoreMesh(core_axis_name="c", subcore_axis_name="s")
  f = pl.kernel(
      _emb_body,
      out_type=jax.ShapeDtypeStruct((N_TOTAL * D,), jnp.float32),
      mesh=mesh,
      scratch_types=[
          pltpu.VMEM((2, XBLK), jnp.int32),
          pltpu.VMEM((2, OVBLK), jnp.float32),
          pltpu.VMEM((D * L,), jnp.float32),
          pltpu.SemaphoreType.DMA,
          pltpu.SemaphoreType.DMA((2,)),
      ],
      compiler_params=pltpu.CompilerParams(needs_layout_passes=False),
  )
  return f(x_flat, tab_t)


def kernel(x, table):
  x_q = (
      x.T.reshape(NJT, 8, NIT, 128)
      .transpose(0, 2, 1, 3)
      .reshape(-1)
  )
  tab_t = jnp.zeros((D, L), jnp.float32).at[:, : table.shape[0]].set(table.T)
  out_q = _emb_call(x_q, tab_t.reshape(-1))
  return (
      out_q.reshape(COLS, NIT, D, 128)
      .transpose(1, 3, 0, 2)
      .reshape(ROWS, COLS, D)
  )
